# baseline (speedup 1.0000x reference)
import jax
import jax.numpy as jnp
import numpy as np
from jax.experimental import pallas as pl
from jax.experimental.pallas import tpu as pltpu

N = 256
HR = 512
H1 = 4
C1 = HR // H1
H2 = 8
C2 = (2 * HR) // H2


def _mm(a, b):
    return jnp.dot(a, b, preferred_element_type=jnp.float32)


def _tconv_dense(x, lrT, q, k, v, skip, We, heads, ch):
    outs = []
    for h in range(heads):
        sl = slice(h * ch, (h + 1) * ch)
        qh, kh, vh = q[:, sl], k[:, sl], v[:, sl]
        weh = We[:, sl]
        d = jnp.sum(qh * weh, axis=1, keepdims=True)
        S = _mm(qh, kh.T) + d * lrT
        m = jnp.max(S, axis=1, keepdims=True)
        ex = jnp.exp(S - m)
        den = jnp.sum(ex, axis=1, keepdims=True)
        r = jnp.sum(ex * lrT, axis=1, keepdims=True)
        outs.append((_mm(ex, vh) + r * weh) / den)
    return jnp.concatenate(outs, axis=1) + skip


def _gnorm(x, g, b, ms):
    mean = jnp.mean(x, axis=0, keepdims=True)
    out = x - ms * mean
    var = jnp.mean(out * out, axis=0, keepdims=True)
    return g * out * jax.lax.rsqrt(var + 1e-5) + b


def _pipeline_kernel(lr_ref,
                     Wq1_ref, bq1_ref, Wk1_ref, bk1_ref, Wv1_hbm, bv1_ref,
                     We1_ref, Ws1_hbm, bs1_ref, g1_ref, b1_ref, ms1_ref,
                     Wq2_hbm, bq2_ref, Wk2_hbm, bk2_ref, Wv2_hbm, bv2_ref,
                     We2_ref, Ws2_hbm, bs2_ref, g2_ref, b2_ref, ms2_ref,
                     out_ref,
                     wv1_v, ws1_v, wq2_v, wk2_v, wv2_v, ws2_v,
                     sv1, ss1, sq2, sk2, sv2, ss2):
    copies = [
        pltpu.make_async_copy(Wv1_hbm, wv1_v, sv1),
        pltpu.make_async_copy(Ws1_hbm, ws1_v, ss1),
        pltpu.make_async_copy(Wq2_hbm, wq2_v, sq2),
        pltpu.make_async_copy(Wk2_hbm, wk2_v, sk2),
        pltpu.make_async_copy(Wv2_hbm, wv2_v, sv2),
        pltpu.make_async_copy(Ws2_hbm, ws2_v, ss2),
    ]
    for c in copies:
        c.start()
    cv1, cs1, cq2, ck2, cv2, cs2 = copies

    lr = lr_ref[...]
    lrT = lr.T

    q1 = (_mm(lr, Wq1_ref[...]) + bq1_ref[...]) * (1.0 / np.sqrt(float(C1)))
    k1 = _mm(lr, Wk1_ref[...]) + bk1_ref[...]
    cv1.wait()
    v1 = _mm(lr, wv1_v[...]) + bv1_ref[...]
    cs1.wait()
    skip1 = _mm(lr, ws1_v[...]) + bs1_ref[...]
    h1 = _tconv_dense(lr, lrT, q1, k1, v1, skip1, We1_ref[...], H1, C1)
    h1 = _gnorm(h1, g1_ref[...], b1_ref[...], ms1_ref[...])

    cq2.wait()
    q2 = (_mm(h1, wq2_v[...]) + bq2_ref[...]) * (1.0 / np.sqrt(float(C2)))
    ck2.wait()
    k2 = _mm(h1, wk2_v[...]) + bk2_ref[...]
    cv2.wait()
    v2 = _mm(h1, wv2_v[...]) + bv2_ref[...]
    cs2.wait()
    skip2 = _mm(h1, ws2_v[...]) + bs2_ref[...]
    h2 = _tconv_dense(h1, lrT, q2, k2, v2, skip2, We2_ref[...], H2, C2)
    h2 = _gnorm(h2, g2_ref[...], b2_ref[...], ms2_ref[...])

    nrm = jnp.sqrt(jnp.sum(h2 * h2, axis=1, keepdims=True))
    out_ref[...] = h2 / nrm


@jax.jit
def _run(lr, Wq1, bq1, Wk1, bk1, Wv1, bv1, We1, Ws1, bs1, g1, b1, ms1,
         Wq2, bq2, Wk2, bk2, Wv2, bv2, We2, Ws2, bs2, g2, b2, ms2):
    args = (lr,
            Wq1, bq1, Wk1, bk1, Wv1, bv1,
            We1, Ws1, bs1, g1, b1, ms1,
            Wq2, bq2, Wk2, bk2, Wv2, bv2,
            We2, Ws2, bs2, g2, b2, ms2)
    hbm_idx = {5, 8, 13, 15, 17, 20}
    in_specs = [
        pl.BlockSpec(memory_space=pl.ANY if i in hbm_idx else pltpu.VMEM)
        for i in range(len(args))
    ]
    w1 = pltpu.VMEM((N, HR), jnp.float32)
    w2 = pltpu.VMEM((HR, 2 * HR), jnp.float32)
    return pl.pallas_call(
        _pipeline_kernel,
        out_shape=jax.ShapeDtypeStruct((N, 2 * HR), jnp.float32),
        in_specs=in_specs,
        out_specs=pl.BlockSpec(memory_space=pltpu.VMEM),
        scratch_shapes=[w1, w1, w2, w2, w2, w2] + [pltpu.SemaphoreType.DMA] * 6,
    )(*args)


def kernel(lr, Wq1, bq1, Wk1, bk1, Wv1, bv1, We1, Ws1, bs1, g1, b1, ms1,
           Wq2, bq2, Wk2, bk2, Wv2, bv2, We2, Ws2, bs2, g2, b2, ms2,
           edge_index):
    del edge_index
    return _run(lr, Wq1, bq1, Wk1, bk1, Wv1, bv1, We1, Ws1, bs1, g1, b1, ms1,
                Wq2, bq2, Wk2, bk2, Wv2, bv2, We2, Ws2, bs2, g2, b2, ms2)


# speedup vs baseline: 1.1361x; 1.1361x over previous
import jax
import jax.numpy as jnp
import numpy as np
from jax.experimental import pallas as pl
from jax.experimental.pallas import tpu as pltpu

N = 256
HR = 512
H1 = 4
C1 = HR // H1
H2 = 8
C2 = (2 * HR) // H2


def _mm(a, b):
    return jnp.dot(a, b, preferred_element_type=jnp.float32)


def _tconv_dense(x, lrT, q, k, v, skip, We, heads, ch):
    outs = []
    for h in range(heads):
        sl = slice(h * ch, (h + 1) * ch)
        qh, kh, vh = q[:, sl], k[:, sl], v[:, sl]
        weh = We[:, sl]
        d = jnp.sum(qh * weh, axis=1, keepdims=True)
        S = _mm(qh, kh.T) + d * lrT
        m = jnp.max(S, axis=1, keepdims=True)
        ex = jnp.exp(S - m)
        den = jnp.sum(ex, axis=1, keepdims=True)
        r = jnp.sum(ex * lrT, axis=1, keepdims=True)
        outs.append((_mm(ex, vh) + r * weh) / den)
    return jnp.concatenate(outs, axis=1) + skip


def _gnorm(x, g, b, ms):
    mean = jnp.mean(x, axis=0, keepdims=True)
    out = x - ms * mean
    var = jnp.mean(out * out, axis=0, keepdims=True)
    return g * out * jax.lax.rsqrt(var + 1e-5) + b


def _pipeline_kernel(lr_ref,
                     Wq1_ref, bq1_ref, Wk1_ref, bk1_ref, Wv1_ref, bv1_ref,
                     We1_ref, Ws1_ref, bs1_ref, g1_ref, b1_ref, ms1_ref,
                     Wq2_hbm, bq2_ref, Wk2_hbm, bk2_ref, Wv2_hbm, bv2_ref,
                     We2_ref, Ws2_hbm, bs2_ref, g2_ref, b2_ref, ms2_ref,
                     out_ref,
                     wq2_v, wk2_v, wv2_v, ws2_v,
                     sq2, sk2, sv2, ss2):
    copies = [
        pltpu.make_async_copy(Wq2_hbm, wq2_v, sq2),
        pltpu.make_async_copy(Wk2_hbm, wk2_v, sk2),
        pltpu.make_async_copy(Wv2_hbm, wv2_v, sv2),
        pltpu.make_async_copy(Ws2_hbm, ws2_v, ss2),
    ]
    for c in copies:
        c.start()
    cq2, ck2, cv2, cs2 = copies

    lr = lr_ref[...]
    lrT = lr.T

    q1 = (_mm(lr, Wq1_ref[...]) + bq1_ref[...]) * (1.0 / np.sqrt(float(C1)))
    k1 = _mm(lr, Wk1_ref[...]) + bk1_ref[...]
    v1 = _mm(lr, Wv1_ref[...]) + bv1_ref[...]
    skip1 = _mm(lr, Ws1_ref[...]) + bs1_ref[...]
    h1 = _tconv_dense(lr, lrT, q1, k1, v1, skip1, We1_ref[...], H1, C1)
    h1 = _gnorm(h1, g1_ref[...], b1_ref[...], ms1_ref[...])

    cq2.wait()
    q2 = (_mm(h1, wq2_v[...]) + bq2_ref[...]) * (1.0 / np.sqrt(float(C2)))
    ck2.wait()
    k2 = _mm(h1, wk2_v[...]) + bk2_ref[...]
    cv2.wait()
    v2 = _mm(h1, wv2_v[...]) + bv2_ref[...]
    cs2.wait()
    skip2 = _mm(h1, ws2_v[...]) + bs2_ref[...]
    h2 = _tconv_dense(h1, lrT, q2, k2, v2, skip2, We2_ref[...], H2, C2)
    h2 = _gnorm(h2, g2_ref[...], b2_ref[...], ms2_ref[...])

    nrm = jnp.sqrt(jnp.sum(h2 * h2, axis=1, keepdims=True))
    out_ref[...] = h2 / nrm


@jax.jit
def _run(lr, Wq1, bq1, Wk1, bk1, Wv1, bv1, We1, Ws1, bs1, g1, b1, ms1,
         Wq2, bq2, Wk2, bk2, Wv2, bv2, We2, Ws2, bs2, g2, b2, ms2):
    args = (lr,
            Wq1, bq1, Wk1, bk1, Wv1, bv1,
            We1, Ws1, bs1, g1, b1, ms1,
            Wq2, bq2, Wk2, bk2, Wv2, bv2,
            We2, Ws2, bs2, g2, b2, ms2)
    hbm_idx = {13, 15, 17, 20}
    in_specs = [
        pl.BlockSpec(memory_space=pl.ANY if i in hbm_idx else pltpu.VMEM)
        for i in range(len(args))
    ]
    w2 = pltpu.VMEM((HR, 2 * HR), jnp.float32)
    return pl.pallas_call(
        _pipeline_kernel,
        out_shape=jax.ShapeDtypeStruct((N, 2 * HR), jnp.float32),
        in_specs=in_specs,
        out_specs=pl.BlockSpec(memory_space=pltpu.VMEM),
        scratch_shapes=[w2, w2, w2, w2] + [pltpu.SemaphoreType.DMA] * 4,
    )(*args)


def kernel(lr, Wq1, bq1, Wk1, bk1, Wv1, bv1, We1, Ws1, bs1, g1, b1, ms1,
           Wq2, bq2, Wk2, bk2, Wv2, bv2, We2, Ws2, bs2, g2, b2, ms2,
           edge_index):
    del edge_index
    return _run(lr, Wq1, bq1, Wk1, bk1, Wv1, bv1, We1, Ws1, bs1, g1, b1, ms1,
                Wq2, bq2, Wk2, bk2, Wv2, bv2, We2, Ws2, bs2, g2, b2, ms2)
